# baseline (speedup 1.0000x reference)
import functools

import jax
import jax.numpy as jnp
from jax import lax
from jax.experimental import pallas as pl
from jax.experimental.pallas import tpu as pltpu
from jax.experimental.pallas import tpu_sc as plsc

D = 1024
NC, NS = 2, 16
NW = NC * NS
LANES = 16
CH = 8
NR = 3


def _emb_kernel(b_sz, t_len):
    tw = t_len // NW
    n_g = tw // CH
    mesh = plsc.VectorSubcoreMesh(
        core_axis_name="c", subcore_axis_name="s", num_cores=NC,
        num_subcores=NS)

    tok_scratch = [pltpu.VMEM((CH, D), jnp.float32)
                   for _ in range(NR * b_sz)]

    @functools.partial(
        pl.kernel,
        out_type=jax.ShapeDtypeStruct((b_sz * t_len, D), jnp.float32),
        mesh=mesh,
        scratch_types=[
            pltpu.VMEM((b_sz * tw,), jnp.int32),
            pltpu.VMEM((CH, D), jnp.float32),
            pltpu.VMEM((CH, D), jnp.float32),
            *tok_scratch,
            *([pltpu.SemaphoreType.DMA] * (2 * NR + 3)),
        ],
    )
    def k(idx_hbm, tok_hbm, pos_hbm, out_hbm,
          idx_v, pos0, pos1, *rest):
        toks = [rest[i * b_sz:(i + 1) * b_sz] for i in range(NR)]
        sg = rest[NR * b_sz:NR * b_sz + NR]
        so = rest[NR * b_sz + NR:NR * b_sz + 2 * NR]
        sp = rest[NR * b_sz + 2 * NR:NR * b_sz + 2 * NR + 2]
        si = rest[NR * b_sz + 2 * NR + 2]
        poss = (pos0, pos1)

        wid = lax.axis_index("s") * NC + lax.axis_index("c")
        tbase = wid * tw
        idx_cps = [
            pltpu.async_copy(idx_hbm.at[pl.ds(b * t_len + tbase, tw)],
                             idx_v.at[pl.ds(b * tw, tw)], si)
            for b in range(b_sz)]

        def start_pos(g):
            return pltpu.async_copy(
                pos_hbm.at[pl.ds(tbase + g * CH, CH)], poss[g % 2],
                sp[g % 2])

        def start_gathers(g):
            cps = []
            for b in range(b_sz):
                idx_sl = idx_v.at[pl.ds(b * tw + g * CH, CH)]
                cps.append(pltpu.async_copy(
                    tok_hbm.at[idx_sl], toks[g % NR][b], sg[g % NR]))
            return cps

        def start_outs(g):
            cps = []
            for b in range(b_sz):
                out_off = b * t_len + tbase + g * CH
                cps.append(pltpu.async_copy(
                    toks[g % NR][b], out_hbm.at[pl.ds(out_off, CH)],
                    so[g % NR]))
            return cps

        pos_cps = [None] * n_g
        g_cps = [None] * n_g
        o_cps = [None] * n_g
        drained = set()
        pos_cps[0] = start_pos(0)
        pos_cps[1] = start_pos(1)
        for cp in idx_cps:
            cp.wait()
        g_cps[0] = start_gathers(0)
        g_cps[1] = start_gathers(1)

        for g in range(n_g):
            for cp in g_cps[g]:
                cp.wait()
            pos_cps[g].wait()
            cur = toks[g % NR]
            pv = poss[g % 2]

            def add_row(r, _):
                for j in range(D // LANES):
                    v = pv[r, pl.ds(j * LANES, LANES)]
                    for b in range(b_sz):
                        plsc.addupdate(
                            cur[b].at[r, pl.ds(j * LANES, LANES)], v)
                return 0

            lax.fori_loop(0, CH, add_row, 0)
            o_cps[g] = start_outs(g)
            if g + 2 < n_g:
                if g >= 1:
                    for cp in o_cps[g - 1]:
                        cp.wait()
                    drained.add(g - 1)
                pos_cps[g + 2] = start_pos(g + 2)
                g_cps[g + 2] = start_gathers(g + 2)
        for g in range(n_g):
            if g not in drained:
                for cp in o_cps[g]:
                    cp.wait()

    return k


@jax.jit
def kernel(x, tok_table, pos_table):
    b, t = x.shape
    idx = x.reshape(-1).astype(jnp.int32)
    out = _emb_kernel(b, t)(idx, tok_table, pos_table)
    return out.reshape(b, t, D)


# speedup vs baseline: 1.0164x; 1.0164x over previous
import functools

import jax
import jax.numpy as jnp
from jax import lax
from jax.experimental import pallas as pl
from jax.experimental.pallas import tpu as pltpu
from jax.experimental.pallas import tpu_sc as plsc

D = 1024
NC, NS = 2, 16
NW = NC * NS
LANES = 16
CH = 8
NR = 3


def _emb_kernel(b_sz, t_len):
    tw = t_len // NW
    n_g = tw // CH
    rows_g = b_sz * CH
    mesh = plsc.VectorSubcoreMesh(
        core_axis_name="c", subcore_axis_name="s", num_cores=NC,
        num_subcores=NS)

    @functools.partial(
        pl.kernel,
        out_type=jax.ShapeDtypeStruct((b_sz * t_len, D), jnp.float32),
        mesh=mesh,
        scratch_types=[
            pltpu.VMEM((b_sz * tw,), jnp.int32),
            pltpu.VMEM((CH, D), jnp.float32),
            pltpu.VMEM((CH, D), jnp.float32),
            *([pltpu.VMEM((rows_g, D), jnp.float32)] * NR),
            *([pltpu.SemaphoreType.DMA] * (2 * NR + 3)),
        ],
    )
    def k(idx_hbm, tok_hbm, pos_hbm, out_hbm,
          idx_v, pos0, pos1, *rest):
        toks = rest[0:NR]
        sg = rest[NR:2 * NR]
        so = rest[2 * NR:3 * NR]
        sp = rest[3 * NR:3 * NR + 2]
        si = rest[3 * NR + 2]
        poss = (pos0, pos1)

        wid = lax.axis_index("s") * NC + lax.axis_index("c")
        tbase = wid * tw
        idx_cp = pltpu.async_copy(
            idx_hbm.at[pl.ds(tbase * b_sz, tw * b_sz)], idx_v, si)

        lane = lax.iota(jnp.int32, LANES)
        stat = (lax.rem(lane, b_sz) * t_len
                + lax.div(lane, b_sz))

        def start_pos(g):
            return pltpu.async_copy(
                pos_hbm.at[pl.ds(tbase + g * CH, CH)], poss[g % 2],
                sp[g % 2])

        def start_gather(g):
            idx_sl = idx_v.at[pl.ds(g * rows_g, rows_g)]
            return pltpu.async_copy(
                tok_hbm.at[idx_sl], toks[g % NR], sg[g % NR])

        def start_outs(g):
            cps = []
            for h in range(rows_g // LANES):
                vec = stat + (tbase + g * CH + h * (LANES // b_sz))
                cps.append(pltpu.async_copy(
                    toks[g % NR].at[pl.ds(h * LANES, LANES)],
                    out_hbm.at[vec], so[g % NR]))
            return cps

        pos_cps = [None] * n_g
        g_cps = [None] * n_g
        o_cps = [None] * n_g
        drained = set()
        pos_cps[0] = start_pos(0)
        pos_cps[1] = start_pos(1)
        idx_cp.wait()
        g_cps[0] = start_gather(0)
        g_cps[1] = start_gather(1)

        for g in range(n_g):
            g_cps[g].wait()
            pos_cps[g].wait()
            cur = toks[g % NR]
            pv = poss[g % 2]

            def add_row(rt, _):
                for j in range(D // LANES):
                    v = pv[rt, pl.ds(j * LANES, LANES)]
                    for b in range(b_sz):
                        plsc.addupdate(
                            cur.at[rt * b_sz + b,
                                   pl.ds(j * LANES, LANES)], v)
                return 0

            lax.fori_loop(0, CH, add_row, 0)
            o_cps[g] = start_outs(g)
            if g + 2 < n_g:
                if g >= 1:
                    for cp in o_cps[g - 1]:
                        cp.wait()
                    drained.add(g - 1)
                pos_cps[g + 2] = start_pos(g + 2)
                g_cps[g + 2] = start_gather(g + 2)
        for g in range(n_g):
            if g not in drained:
                for cp in o_cps[g]:
                    cp.wait()

    return k


@jax.jit
def kernel(x, tok_table, pos_table):
    b, t = x.shape
    idx = x.astype(jnp.int32).T.reshape(-1)
    out = _emb_kernel(b, t)(idx, tok_table, pos_table)
    return out.reshape(b, t, D)
